# baseline (speedup 1.0000x reference)
import functools
import math

import jax
import jax.numpy as jnp
from jax import lax
from jax.experimental import pallas as pl
from jax.experimental.pallas import tpu as pltpu
from jax.experimental.pallas import tpu_sc as plsc

D = 128
SCALE = math.sqrt(float(D))
NC = 2
NS = 16
NW = NC * NS
B = 1024 * 200
B_PER_W = B // NW
CHUNK = 128
NCHUNK = B_PER_W // CHUNK
NIN = 3
NOUT = 3
import math as _math
UNROLL = NIN * NOUT // _math.gcd(NIN, NOUT)


def _emb_body(x_hbm, table_hbm, out_hbm, idx_v, *scratch):
    in_bufs = scratch[:NIN]
    out_bufs = scratch[NIN:NIN + NOUT]
    in_sems = scratch[NIN + NOUT:2 * NIN + NOUT]
    out_sems = scratch[2 * NIN + NOUT:2 * NIN + 2 * NOUT]

    c = lax.axis_index("c")
    s = lax.axis_index("s")
    wid = s * NC + c
    base = wid * B_PER_W

    pltpu.sync_copy(x_hbm.at[wid], idx_v)

    def gather(chunk, bi):
        return pltpu.make_async_copy(
            table_hbm.at[idx_v.at[chunk]], in_bufs[bi], in_sems[bi])

    def put(chunk, bo):
        return pltpu.make_async_copy(
            out_bufs[bo], out_hbm.at[pl.ds(base + chunk * CHUNK, CHUNK)],
            out_sems[bo])

    for bi in range(NIN):
        gather(jnp.int32(bi), bi).start()

    def slot(chunk, bi, bo):
        gather(chunk, bi).wait()
        @pl.when(chunk >= NOUT)
        def _():
            put(chunk - NOUT, bo).wait()

        def row(r, carry):
            for cc in range(D // 16):
                out_bufs[bo][r, pl.ds(cc * 16, 16)] = (
                    in_bufs[bi][r, pl.ds(cc * 16, 16)] * SCALE)
            return carry
        lax.fori_loop(0, CHUNK, row, 0)

        @pl.when(chunk + NIN < NCHUNK)
        def _():
            gather(chunk + NIN, bi).start()

        put(chunk, bo).start()

    full, rem = divmod(NCHUNK, UNROLL)

    def outer(j, carry):
        for k in range(UNROLL):
            slot(j * UNROLL + k, k % NIN, k % NOUT)
        return carry
    lax.fori_loop(0, full, outer, 0)
    for k in range(rem):
        chunk = full * UNROLL + k
        slot(jnp.int32(chunk), chunk % NIN, chunk % NOUT)

    for chunk in range(NCHUNK - NOUT, NCHUNK):
        put(jnp.int32(chunk), chunk % NOUT).wait()


def kernel(x, table):
    mesh = plsc.VectorSubcoreMesh(core_axis_name="c", subcore_axis_name="s")
    scratch = (
        [pltpu.VMEM((NCHUNK, CHUNK), jnp.int32)]
        + [pltpu.VMEM((CHUNK, D), jnp.float32) for _ in range(NIN + NOUT)]
        + [pltpu.SemaphoreType.DMA for _ in range(NIN + NOUT)]
    )
    run = functools.partial(
        pl.kernel,
        mesh=mesh,
        out_type=jax.ShapeDtypeStruct((B, D), jnp.float32),
        scratch_types=scratch,
    )(_emb_body)
    x3d = x.reshape(NW, NCHUNK, CHUNK).astype(jnp.int32)
    out = run(x3d, table)
    return out.reshape(x.shape[0], x.shape[1], D)


# speedup vs baseline: 1.0132x; 1.0132x over previous
import functools
import math

import jax
import jax.numpy as jnp
from jax import lax
from jax.experimental import pallas as pl
from jax.experimental.pallas import tpu as pltpu
from jax.experimental.pallas import tpu_sc as plsc

D = 128
SCALE = math.sqrt(float(D))
NC = 2
NS = 16
NW = NC * NS
B = 1024 * 200
B_PER_W = B // NW
CHUNK = 64
NCHUNK = B_PER_W // CHUNK
NIN = 6
NOUT = 6
import math as _math
UNROLL = NIN * NOUT // _math.gcd(NIN, NOUT)


def _emb_body(x_hbm, table_hbm, out_hbm, idx_v, *scratch):
    in_bufs = scratch[:NIN]
    out_bufs = scratch[NIN:NIN + NOUT]
    in_sems = scratch[NIN + NOUT:2 * NIN + NOUT]
    out_sems = scratch[2 * NIN + NOUT:2 * NIN + 2 * NOUT]

    c = lax.axis_index("c")
    s = lax.axis_index("s")
    wid = s * NC + c
    base = wid * B_PER_W

    pltpu.sync_copy(x_hbm.at[wid], idx_v)

    def gather(chunk, bi):
        return pltpu.make_async_copy(
            table_hbm.at[idx_v.at[chunk]], in_bufs[bi], in_sems[bi])

    def put(chunk, bo):
        return pltpu.make_async_copy(
            out_bufs[bo], out_hbm.at[pl.ds(base + chunk * CHUNK, CHUNK)],
            out_sems[bo])

    for bi in range(NIN):
        gather(jnp.int32(bi), bi).start()

    def slot(chunk, bi, bo):
        gather(chunk, bi).wait()
        @pl.when(chunk >= NOUT)
        def _():
            put(chunk - NOUT, bo).wait()

        def row(r, carry):
            for cc in range(D // 16):
                out_bufs[bo][r, pl.ds(cc * 16, 16)] = (
                    in_bufs[bi][r, pl.ds(cc * 16, 16)] * SCALE)
            return carry
        lax.fori_loop(0, CHUNK, row, 0)

        @pl.when(chunk + NIN < NCHUNK)
        def _():
            gather(chunk + NIN, bi).start()

        put(chunk, bo).start()

    full, rem = divmod(NCHUNK, UNROLL)

    def outer(j, carry):
        for k in range(UNROLL):
            slot(j * UNROLL + k, k % NIN, k % NOUT)
        return carry
    lax.fori_loop(0, full, outer, 0)
    for k in range(rem):
        chunk = full * UNROLL + k
        slot(jnp.int32(chunk), chunk % NIN, chunk % NOUT)

    for chunk in range(NCHUNK - NOUT, NCHUNK):
        put(jnp.int32(chunk), chunk % NOUT).wait()


def kernel(x, table):
    mesh = plsc.VectorSubcoreMesh(core_axis_name="c", subcore_axis_name="s")
    scratch = (
        [pltpu.VMEM((NCHUNK, CHUNK), jnp.int32)]
        + [pltpu.VMEM((CHUNK, D), jnp.float32) for _ in range(NIN + NOUT)]
        + [pltpu.SemaphoreType.DMA for _ in range(NIN + NOUT)]
    )
    run = functools.partial(
        pl.kernel,
        mesh=mesh,
        out_type=jax.ShapeDtypeStruct((B, D), jnp.float32),
        scratch_types=scratch,
    )(_emb_body)
    x3d = x.reshape(NW, NCHUNK, CHUNK).astype(jnp.int32)
    out = run(x3d, table)
    return out.reshape(x.shape[0], x.shape[1], D)
